# baseline (speedup 1.0000x reference)
import jax
import jax.numpy as jnp
from jax.experimental import pallas as pl
from jax.experimental.pallas import tpu as pltpu

_TN = 2048
_SCALE = [1.0]


def _sym_loss_kernel(a_ref, bt_ref, n_ref, out_ref):
    b = pl.program_id(0)
    t = pl.program_id(1)

    @pl.when(jnp.logical_and(b == 0, t == 0))
    def _init():
        out_ref[0, 0] = 0.0

    a = a_ref[0]
    bt = bt_ref[0]
    ab2 = jnp.dot(a, bt, preferred_element_type=jnp.float32)
    nrow = n_ref[0]
    e = nrow - ab2
    m = jnp.min(e, axis=1, keepdims=True)
    acc = jnp.sum(m)

    @pl.when(t == 0)
    def _add_norms():
        out_ref[0, 0] += jnp.sum(nrow)

    out_ref[0, 0] += acc

    nb = pl.num_programs(0)
    nt = pl.num_programs(1)

    @pl.when(jnp.logical_and(b == nb - 1, t == nt - 1))
    def _finish():
        out_ref[0, 0] = out_ref[0, 0] * _SCALE[0]


def kernel(xyz):
    B, N, _ = xyz.shape
    x = xyz[..., 0]
    y = xyz[..., 1]
    z = xyz[..., 2]
    n = x * x + y * y + z * z
    zeros = jnp.zeros_like(n)
    a = jnp.stack([-2 * x, 2 * y, 2 * z, zeros], axis=-1).astype(jnp.bfloat16)
    bt = jnp.stack([x, y, z, zeros], axis=1).astype(jnp.bfloat16)
    nr = n[:, None, :]

    _SCALE[0] = 2.0 / (B * N)
    total = pl.pallas_call(
        _sym_loss_kernel,
        grid=(B, N // _TN),
        in_specs=[
            pl.BlockSpec((1, _TN, 4), lambda b, t: (b, t, 0)),
            pl.BlockSpec((1, 4, N), lambda b, t: (b, 0, 0)),
            pl.BlockSpec((1, 1, N), lambda b, t: (b, 0, 0)),
        ],
        out_specs=pl.BlockSpec(memory_space=pltpu.SMEM),
        out_shape=jax.ShapeDtypeStruct((1, 1), jnp.float32),
    )(a, bt, nr)
    return total[0, 0]


# speedup vs baseline: 1.0727x; 1.0727x over previous
import jax
import jax.numpy as jnp
from jax.experimental import pallas as pl
from jax.experimental.pallas import tpu as pltpu

_TN = 4096
_SCALE = [1.0]


def _sym_loss_kernel(a_ref, bt_ref, n_ref, out_ref):
    b = pl.program_id(0)
    t = pl.program_id(1)

    @pl.when(jnp.logical_and(b == 0, t == 0))
    def _init():
        out_ref[0, 0] = 0.0

    a = a_ref[0]
    bt = bt_ref[0]
    ab2 = jnp.dot(a, bt, preferred_element_type=jnp.float32)
    nrow = n_ref[0]
    e = nrow - ab2
    m = jnp.min(e, axis=1, keepdims=True)
    acc = jnp.sum(m)

    @pl.when(t == 0)
    def _add_norms():
        out_ref[0, 0] += jnp.sum(nrow)

    out_ref[0, 0] += acc

    nb = pl.num_programs(0)
    nt = pl.num_programs(1)

    @pl.when(jnp.logical_and(b == nb - 1, t == nt - 1))
    def _finish():
        out_ref[0, 0] = out_ref[0, 0] * _SCALE[0]


def kernel(xyz):
    B, N, _ = xyz.shape
    scale = jnp.array([-2.0, 2.0, 2.0], xyz.dtype)
    a = (xyz * scale).astype(jnp.bfloat16)
    xyzT = jnp.swapaxes(xyz, 1, 2)
    bt = xyzT.astype(jnp.bfloat16)
    nr = jnp.sum(xyzT * xyzT, axis=1, keepdims=True)

    _SCALE[0] = 2.0 / (B * N)
    total = pl.pallas_call(
        _sym_loss_kernel,
        grid=(B, N // _TN),
        in_specs=[
            pl.BlockSpec((1, _TN, 3), lambda b, t: (b, t, 0)),
            pl.BlockSpec((1, 3, N), lambda b, t: (b, 0, 0)),
            pl.BlockSpec((1, 1, N), lambda b, t: (b, 0, 0)),
        ],
        out_specs=pl.BlockSpec(memory_space=pltpu.SMEM),
        out_shape=jax.ShapeDtypeStruct((1, 1), jnp.float32),
    )(a, bt, nr)
    return total[0, 0]
